# baseline (speedup 1.0000x reference)
import jax
import jax.numpy as jnp
from jax.experimental import pallas as pl
from jax.experimental.pallas import tpu as pltpu

_N = 100000
_D = 128
_BR = 2000


def _body(c_ref, rgb_f_ref, flow_f_ref, rgb_in, flow_in, counts_ref,
          rgb_out, flow_out):
    i = pl.program_id(0)
    c = c_ref[0]
    rows = i * _BR + jax.lax.broadcasted_iota(jnp.int32, (_BR, 1), 0)
    mask = rows == c
    n = counts_ref[...]
    scale = n / (n + 1.0)
    inv = 1.0 / (n + 1.0)
    rgb = rgb_in[...]
    flow = flow_in[...]
    rgb_out[...] = jnp.where(mask, scale * rgb + inv * rgb_f_ref[...], rgb)
    flow_out[...] = jnp.where(mask, scale * flow + inv * flow_f_ref[...], flow)


def kernel(rgb_feat, flow_feat, pseudo_label, rgb_prototypes, flow_prototypes, counts):
    c = jnp.asarray(pseudo_label, jnp.int32).reshape(1)
    rgb_f = rgb_feat.reshape(1, _D)
    flow_f = flow_feat.reshape(1, _D)
    counts2 = counts.reshape(_N, 1)
    grid = _N // _BR
    out = pl.pallas_call(
        _body,
        grid=(grid,),
        in_specs=[
            pl.BlockSpec(memory_space=pltpu.SMEM),
            pl.BlockSpec((1, _D), lambda i: (0, 0)),
            pl.BlockSpec((1, _D), lambda i: (0, 0)),
            pl.BlockSpec((_BR, _D), lambda i: (i, 0)),
            pl.BlockSpec((_BR, _D), lambda i: (i, 0)),
            pl.BlockSpec((_BR, 1), lambda i: (i, 0)),
        ],
        out_specs=[
            pl.BlockSpec((_BR, _D), lambda i: (i, 0)),
            pl.BlockSpec((_BR, _D), lambda i: (i, 0)),
        ],
        out_shape=[
            jax.ShapeDtypeStruct((_N, _D), jnp.float32),
            jax.ShapeDtypeStruct((_N, _D), jnp.float32),
        ],
    )(c, rgb_f, flow_f, rgb_prototypes, flow_prototypes, counts2)
    return (out[0], out[1])
